# baseline (speedup 1.0000x reference)
import functools

import jax
import jax.numpy as jnp
from jax import lax
from jax.experimental import pallas as pl
from jax.experimental.pallas import tpu as pltpu
from jax.experimental.pallas import tpu_sc as plsc

_CHUNK = 128
_NBUF = 2


def kernel(indices, values):
    B, L = indices.shape
    V, D = values.shape
    N = B * L

    info = plsc.get_sparse_core_info()
    NC, NS = info.num_cores, info.num_subcores
    NW = NC * NS
    per_w = N // NW
    n_chunks = per_w // _CHUNK
    idx3 = indices.reshape(NW, n_chunks, _CHUNK)

    mesh = plsc.VectorSubcoreMesh(core_axis_name="c", subcore_axis_name="s")

    @functools.partial(
        pl.kernel,
        mesh=mesh,
        compiler_params=pltpu.CompilerParams(use_tc_tiling_on_sc=False),
        out_type=jax.ShapeDtypeStruct((N, D), jnp.float32),
        scratch_types=[
            pltpu.VMEM((n_chunks, _CHUNK), jnp.int32),
            pltpu.VMEM((_NBUF, _CHUNK, D), jnp.float32),
            pltpu.SemaphoreType.DMA,
            pltpu.SemaphoreType.DMA,
        ],
    )
    def _gather(idx_hbm, tab_hbm, out_hbm, idx_v, rows_v, sem0, sem1):
        wid = lax.axis_index("s") * NC + lax.axis_index("c")
        base = wid * per_w
        sems = (sem0, sem1)

        pltpu.sync_copy(idx_hbm.at[wid], idx_v)

        def start(c, b):
            pltpu.async_copy(tab_hbm.at[idx_v.at[c]], rows_v.at[b], sems[b])

        def wait(c, b):
            pltpu.make_async_copy(
                tab_hbm.at[idx_v.at[c]], rows_v.at[b], sems[b]
            ).wait()

        def drain(c, b):
            wait(c, b)
            pltpu.sync_copy(
                rows_v.at[b], out_hbm.at[pl.ds(base + c * _CHUNK, _CHUNK)]
            )

        for b in range(_NBUF):
            start(b, b)

        def body(i, carry):
            for b in range(_NBUF):
                c = _NBUF * i + b
                drain(c, b)
                start(c + _NBUF, b)
            return carry

        lax.fori_loop(0, n_chunks // _NBUF - 1, body, 0)

        for b in range(_NBUF):
            drain(n_chunks - _NBUF + b, b)

    out = _gather(idx3, values)
    return out.reshape(B, L, D)


# speedup vs baseline: 1.0228x; 1.0228x over previous
import functools

import jax
import jax.numpy as jnp
from jax import lax
from jax.experimental import pallas as pl
from jax.experimental.pallas import tpu as pltpu
from jax.experimental.pallas import tpu_sc as plsc

_CHUNK = 128
_NBUF = 2


def kernel(indices, values):
    B, L = indices.shape
    V, D = values.shape
    N = B * L

    info = plsc.get_sparse_core_info()
    NC, NS = info.num_cores, info.num_subcores
    NW = NC * NS
    per_w = N // NW
    n_chunks = per_w // _CHUNK
    idx3 = indices.reshape(NW, n_chunks, _CHUNK)

    mesh = plsc.VectorSubcoreMesh(core_axis_name="c", subcore_axis_name="s")

    @functools.partial(
        pl.kernel,
        mesh=mesh,
        compiler_params=pltpu.CompilerParams(use_tc_tiling_on_sc=False),
        out_type=jax.ShapeDtypeStruct((N, D), jnp.float32),
        scratch_types=[
            pltpu.VMEM((n_chunks, _CHUNK), jnp.int32),
            pltpu.VMEM((per_w, D), jnp.float32),
            pltpu.SemaphoreType.DMA,
        ],
    )
    def _gather(idx_hbm, tab_hbm, out_hbm, idx_v, rows_v, gsem):
        wid = lax.axis_index("s") * NC + lax.axis_index("c")
        base = wid * per_w

        pltpu.sync_copy(idx_hbm.at[wid], idx_v)

        def fire(c, carry):
            pltpu.async_copy(
                tab_hbm.at[idx_v.at[c]],
                rows_v.at[pl.ds(c * _CHUNK, _CHUNK)],
                gsem,
            )
            return carry

        lax.fori_loop(0, n_chunks, fire, 0)

        pltpu.make_async_copy(tab_hbm.at[pl.ds(0, per_w)], rows_v, gsem).wait()

        pltpu.sync_copy(rows_v, out_hbm.at[pl.ds(base, per_w)])

    out = _gather(idx3, values)
    return out.reshape(B, L, D)


# speedup vs baseline: 1.2289x; 1.2015x over previous
import functools

import jax
import jax.numpy as jnp
from jax import lax
from jax.experimental import pallas as pl
from jax.experimental.pallas import tpu as pltpu
from jax.experimental.pallas import tpu_sc as plsc


def kernel(indices, values):
    B, L = indices.shape
    V, D = values.shape

    info = plsc.get_sparse_core_info()
    NC, NS = info.num_cores, info.num_subcores
    NW = NC * NS
    b_per_sub = B // NW

    mesh = plsc.VectorSubcoreMesh(core_axis_name="c", subcore_axis_name="s")

    @functools.partial(
        pl.kernel,
        mesh=mesh,
        compiler_params=pltpu.CompilerParams(
            use_tc_tiling_on_sc=False, needs_layout_passes=False
        ),
        out_type=jax.ShapeDtypeStruct((L, D, B), jnp.float32),
        scratch_types=[
            pltpu.VMEM((L, b_per_sub), jnp.int32),
            pltpu.VMEM((2, b_per_sub, D), jnp.float32),
            pltpu.VMEM((D, b_per_sub), jnp.float32),
            pltpu.SemaphoreType.DMA,
            pltpu.SemaphoreType.DMA,
        ],
    )
    def _gath(w16, idxT, outT, idx_v, rows_v, stg_v, g0, g1):
        wid = lax.axis_index("s") * NC + lax.axis_index("c")
        b0 = wid * b_per_sub
        gsems = (g0, g1)
        iota16 = lax.iota(jnp.int32, 16)

        pltpu.sync_copy(idxT.at[:, pl.ds(b0, b_per_sub)], idx_v)

        def g_start(l, b):
            pltpu.async_copy(w16.at[idx_v.at[l]], rows_v.at[b], gsems[b])

        def g_wait(b):
            pltpu.make_async_copy(
                w16.at[pl.ds(0, b_per_sub)], rows_v.at[b], gsems[b]
            ).wait()

        g_start(0, 0)
        g_start(1, 1)

        def body(i, carry):
            for b in range(2):
                l = 2 * i + b
                g_wait(b)
                for c in range(D):
                    for m in range(b_per_sub // 16):
                        v = plsc.load_gather(
                            rows_v.at[b],
                            [iota16 + 16 * m, jnp.full((16,), c, jnp.int32)],
                        )
                        stg_v[c, pl.ds(16 * m, 16)] = v

                @pl.when(l + 2 < L)
                def _():
                    g_start(l + 2, b)

                pltpu.sync_copy(stg_v, outT.at[l, :, pl.ds(b0, b_per_sub)])
            return carry

        lax.fori_loop(0, L // 2, body, 0)

    outT = _gath(values, indices.T)
    return jnp.transpose(outT, (2, 0, 1))


# speedup vs baseline: 1.2708x; 1.0341x over previous
import functools

import jax
import jax.numpy as jnp
from jax import lax
from jax.experimental import pallas as pl
from jax.experimental.pallas import tpu as pltpu
from jax.experimental.pallas import tpu_sc as plsc

_BK = 1024


def kernel(indices, values):
    B, L = indices.shape
    V, D = values.shape

    info = plsc.get_sparse_core_info()
    NC, NS = info.num_cores, info.num_subcores
    NW = NC * NS

    n_full = V // _BK
    tail = V - n_full * _BK
    wrows_blk = _BK * D // 128
    per_sub = n_full // NW
    extra = n_full - per_sub * NW

    mesh = plsc.VectorSubcoreMesh(core_axis_name="c", subcore_axis_name="s")

    @functools.partial(
        pl.kernel,
        mesh=mesh,
        compiler_params=pltpu.CompilerParams(needs_layout_passes=False),
        out_type=jax.ShapeDtypeStruct((V * D // 128, 128), jnp.float32),
        scratch_types=[
            pltpu.VMEM((2, D, _BK), jnp.float32),
            pltpu.VMEM((2, wrows_blk, 128), jnp.float32),
            pltpu.SemaphoreType.DMA,
            pltpu.SemaphoreType.DMA,
            pltpu.SemaphoreType.DMA,
            pltpu.SemaphoreType.DMA,
        ],
    )
    def _pack(tabT, w_hbm, src_v, stg_v, rs0, rs1, ws0, ws1):
        wid = lax.axis_index("s") * NC + lax.axis_index("c")
        kcount = jnp.where(wid < extra, per_sub + 1, per_sub)
        rsems = (rs0, rs1)
        wsems = (ws0, ws1)
        iota16 = lax.iota(jnp.int32, 16)

        def rd_start(j, b):
            off = pl.multiple_of(j * _BK, _BK)
            pltpu.async_copy(tabT.at[:, pl.ds(off, _BK)], src_v.at[b], rsems[b])

        def rd_wait(b):
            pltpu.make_async_copy(
                tabT.at[:, pl.ds(0, _BK)], src_v.at[b], rsems[b]
            ).wait()

        def wr_start(j, b):
            off = pl.multiple_of(j * wrows_blk, wrows_blk)
            pltpu.async_copy(
                stg_v.at[b], w_hbm.at[pl.ds(off, wrows_blk)], wsems[b]
            )

        def wr_wait(b):
            pltpu.make_async_copy(
                stg_v.at[b], w_hbm.at[pl.ds(0, wrows_blk)], wsems[b]
            ).wait()

        def transpose_block(b):
            def tr(i, carry):
                v = plsc.load_gather(
                    src_v.at[b], [iota16, jnp.full((16,), 0, jnp.int32) + i]
                )
                stg_v[b, i >> 3, pl.ds((i & 7) * 16, 16)] = v
                return carry

            lax.fori_loop(0, _BK, tr, 0, unroll=8)

        rd_start(wid, 0)

        def body(k, carry):
            j = wid + NW * k
            for b in range(2):
                @pl.when(k % 2 == b)
                def _(b=b, j=j, k=k):
                    rd_wait(b)

                    @pl.when(k + 1 < kcount)
                    def _():
                        rd_start(j + NW, 1 - b)

                    @pl.when(k >= 2)
                    def _():
                        wr_wait(b)

                    transpose_block(b)
                    wr_start(j, b)

            return carry

        lax.fori_loop(0, kcount, body, 0)

        @pl.when(kcount >= 2)
        def _():
            wr_wait(0)
            wr_wait(1)

        @pl.when(kcount == 1)
        def _():
            wr_wait(0)

    b_per_sub = B // NW

    @functools.partial(
        pl.kernel,
        mesh=mesh,
        compiler_params=pltpu.CompilerParams(
            use_tc_tiling_on_sc=False, needs_layout_passes=False
        ),
        out_type=jax.ShapeDtypeStruct((L, D, B), jnp.float32),
        scratch_types=[
            pltpu.VMEM((L, b_per_sub), jnp.int32),
            pltpu.VMEM((2, b_per_sub, D), jnp.float32),
            pltpu.VMEM((D, b_per_sub), jnp.float32),
            pltpu.VMEM((tail, D), jnp.float32),
            pltpu.SemaphoreType.DMA,
            pltpu.SemaphoreType.DMA,
        ],
    )
    def _gath(w16, idxT, tail_hbm, outT, idx_v, rows_v, stg_v, tail_v, g0, g1):
        wid = lax.axis_index("s") * NC + lax.axis_index("c")
        b0 = wid * b_per_sub
        gsems = (g0, g1)
        iota16 = lax.iota(jnp.int32, 16)

        pltpu.sync_copy(idxT.at[:, pl.ds(b0, b_per_sub)], idx_v)
        pltpu.sync_copy(tail_hbm, tail_v)

        def g_start(l, b):
            pltpu.async_copy(w16.at[idx_v.at[l]], rows_v.at[b], gsems[b])

        def g_wait(b):
            pltpu.make_async_copy(
                w16.at[pl.ds(0, b_per_sub)], rows_v.at[b], gsems[b]
            ).wait()

        g_start(0, 0)
        g_start(1, 1)

        def body(i, carry):
            for b in range(2):
                l = 2 * i + b
                g_wait(b)
                for m in range(b_per_sub // 16):
                    iv = idx_v[l, pl.ds(16 * m, 16)]
                    msk = iv >= n_full * _BK

                    @pl.when(jnp.max(iv) >= n_full * _BK)
                    def _(iv=iv, msk=msk, m=m, b=b):
                        ti = jnp.where(msk, iv - n_full * _BK, 0)
                        for c in range(D):
                            tv = plsc.load_gather(
                                tail_v, [ti, jnp.full((16,), c, jnp.int32)]
                            )
                            plsc.store_scatter(
                                rows_v.at[b],
                                [iota16 + 16 * m,
                                 jnp.full((16,), c, jnp.int32)],
                                tv, mask=msk,
                            )

                for c in range(D):
                    for m in range(b_per_sub // 16):
                        v = plsc.load_gather(
                            rows_v.at[b],
                            [iota16 + 16 * m, jnp.full((16,), c, jnp.int32)],
                        )
                        stg_v[c, pl.ds(16 * m, 16)] = v

                @pl.when(l + 2 < L)
                def _(l=l, b=b):
                    g_start(l + 2, b)

                pltpu.sync_copy(stg_v, outT.at[l, :, pl.ds(b0, b_per_sub)])
            return carry

        lax.fori_loop(0, L // 2, body, 0)

    w = _pack(values.T)
    outT = _gath(w.reshape(V, D), indices.T, values[n_full * _BK:, :])
    return jnp.transpose(outT, (2, 0, 1))


# speedup vs baseline: 1.8576x; 1.4618x over previous
import functools

import jax
import jax.numpy as jnp
from jax import lax
from jax.experimental import pallas as pl
from jax.experimental.pallas import tpu as pltpu
from jax.experimental.pallas import tpu_sc as plsc

_BK = 1024


def kernel(indices, values):
    B, L = indices.shape
    V, D = values.shape

    info = plsc.get_sparse_core_info()
    NC, NS = info.num_cores, info.num_subcores
    NW = NC * NS

    n_full = V // _BK
    tail = V - n_full * _BK
    wrows_blk = _BK * D // 128
    per_sub = n_full // NW
    extra = n_full - per_sub * NW

    mesh = plsc.VectorSubcoreMesh(core_axis_name="c", subcore_axis_name="s")

    @functools.partial(
        pl.kernel,
        mesh=mesh,
        compiler_params=pltpu.CompilerParams(needs_layout_passes=False),
        out_type=jax.ShapeDtypeStruct((V * D // 128, 128), jnp.float32),
        scratch_types=[
            pltpu.VMEM((2, D, _BK), jnp.float32),
            pltpu.VMEM((2, wrows_blk, 128), jnp.float32),
            pltpu.SemaphoreType.DMA,
            pltpu.SemaphoreType.DMA,
            pltpu.SemaphoreType.DMA,
            pltpu.SemaphoreType.DMA,
        ],
    )
    def _pack(tabT, w_hbm, src_v, stg_v, rs0, rs1, ws0, ws1):
        wid = lax.axis_index("s") * NC + lax.axis_index("c")
        kcount = jnp.where(wid < extra, per_sub + 1, per_sub)
        rsems = (rs0, rs1)
        wsems = (ws0, ws1)
        iota16 = lax.iota(jnp.int32, 16)

        def rd_start(j, b):
            off = pl.multiple_of(j * _BK, _BK)
            pltpu.async_copy(tabT.at[:, pl.ds(off, _BK)], src_v.at[b], rsems[b])

        def rd_wait(b):
            pltpu.make_async_copy(
                tabT.at[:, pl.ds(0, _BK)], src_v.at[b], rsems[b]
            ).wait()

        def wr_start(j, b):
            off = pl.multiple_of(j * wrows_blk, wrows_blk)
            pltpu.async_copy(
                stg_v.at[b], w_hbm.at[pl.ds(off, wrows_blk)], wsems[b]
            )

        def wr_wait(b):
            pltpu.make_async_copy(
                stg_v.at[b], w_hbm.at[pl.ds(0, wrows_blk)], wsems[b]
            ).wait()

        def transpose_block(b):
            @plsc.parallel_loop(0, _BK, unroll=16)
            def _tr(i):
                v = plsc.load_gather(
                    src_v.at[b], [iota16, jnp.full((16,), 0, jnp.int32) + i]
                )
                stg_v[b, i >> 3, pl.ds((i & 7) * 16, 16)] = v

        rd_start(wid, 0)

        def body(k, carry):
            j = wid + NW * k
            for b in range(2):
                @pl.when(k % 2 == b)
                def _(b=b, j=j, k=k):
                    rd_wait(b)

                    @pl.when(k + 1 < kcount)
                    def _():
                        rd_start(j + NW, 1 - b)

                    @pl.when(k >= 2)
                    def _():
                        wr_wait(b)

                    transpose_block(b)
                    wr_start(j, b)

            return carry

        lax.fori_loop(0, kcount, body, 0)

        @pl.when(kcount >= 2)
        def _():
            wr_wait(0)
            wr_wait(1)

        @pl.when(kcount == 1)
        def _():
            wr_wait(0)

    b_per_sub = B // NW

    @functools.partial(
        pl.kernel,
        mesh=mesh,
        compiler_params=pltpu.CompilerParams(
            use_tc_tiling_on_sc=False, needs_layout_passes=False
        ),
        out_type=jax.ShapeDtypeStruct((L, D, B), jnp.float32),
        scratch_types=[
            pltpu.VMEM((L, b_per_sub), jnp.int32),
            pltpu.VMEM((2, b_per_sub, D), jnp.float32),
            pltpu.VMEM((D, b_per_sub), jnp.float32),
            pltpu.VMEM((tail, D), jnp.float32),
            pltpu.SemaphoreType.DMA,
            pltpu.SemaphoreType.DMA,
        ],
    )
    def _gath(w16, idxT, tail_hbm, outT, idx_v, rows_v, stg_v, tail_v, g0, g1):
        wid = lax.axis_index("s") * NC + lax.axis_index("c")
        b0 = wid * b_per_sub
        gsems = (g0, g1)
        iota16 = lax.iota(jnp.int32, 16)

        pltpu.sync_copy(idxT.at[:, pl.ds(b0, b_per_sub)], idx_v)
        pltpu.sync_copy(tail_hbm, tail_v)

        def g_start(l, b):
            pltpu.async_copy(w16.at[idx_v.at[l]], rows_v.at[b], gsems[b])

        def g_wait(b):
            pltpu.make_async_copy(
                w16.at[pl.ds(0, b_per_sub)], rows_v.at[b], gsems[b]
            ).wait()

        g_start(0, 0)
        g_start(1, 1)

        def body(i, carry):
            for b in range(2):
                l = 2 * i + b
                g_wait(b)
                for m in range(b_per_sub // 16):
                    iv = idx_v[l, pl.ds(16 * m, 16)]
                    msk = iv >= n_full * _BK

                    @pl.when(jnp.max(iv) >= n_full * _BK)
                    def _(iv=iv, msk=msk, m=m, b=b):
                        ti = jnp.where(msk, iv - n_full * _BK, 0)
                        for c in range(D):
                            tv = plsc.load_gather(
                                tail_v, [ti, jnp.full((16,), c, jnp.int32)]
                            )
                            plsc.store_scatter(
                                rows_v.at[b],
                                [iota16 + 16 * m,
                                 jnp.full((16,), c, jnp.int32)],
                                tv, mask=msk,
                            )

                for c in range(D):
                    for m in range(b_per_sub // 16):
                        v = plsc.load_gather(
                            rows_v.at[b],
                            [iota16 + 16 * m, jnp.full((16,), c, jnp.int32)],
                        )
                        stg_v[c, pl.ds(16 * m, 16)] = v

                @pl.when(l + 2 < L)
                def _(l=l, b=b):
                    g_start(l + 2, b)

                pltpu.sync_copy(stg_v, outT.at[l, :, pl.ds(b0, b_per_sub)])
            return carry

        lax.fori_loop(0, L // 2, body, 0)

    w = _pack(values.T)
    outT = _gath(w.reshape(V, D), indices.T, values[n_full * _BK:, :])
    return jnp.transpose(outT, (2, 0, 1))


# speedup vs baseline: 2.3204x; 1.2491x over previous
import functools

import jax
import jax.numpy as jnp
from jax import lax
from jax.experimental import pallas as pl
from jax.experimental.pallas import tpu as pltpu
from jax.experimental.pallas import tpu_sc as plsc

_BK = 1024


def kernel(indices, values):
    B, L = indices.shape
    V, D = values.shape

    info = plsc.get_sparse_core_info()
    NC, NS = info.num_cores, info.num_subcores
    NW = NC * NS

    n_full = V // _BK
    tail = V - n_full * _BK
    wrows_blk = _BK * D // 128
    per_sub = n_full // NW
    extra = n_full - per_sub * NW

    mesh = plsc.VectorSubcoreMesh(core_axis_name="c", subcore_axis_name="s")

    @functools.partial(
        pl.kernel,
        mesh=mesh,
        compiler_params=pltpu.CompilerParams(needs_layout_passes=False),
        out_type=jax.ShapeDtypeStruct((V * D // 128, 128), jnp.float32),
        scratch_types=[
            pltpu.VMEM((2, D, _BK), jnp.float32),
            pltpu.VMEM((2, wrows_blk, 128), jnp.float32),
            pltpu.SemaphoreType.DMA,
            pltpu.SemaphoreType.DMA,
            pltpu.SemaphoreType.DMA,
            pltpu.SemaphoreType.DMA,
        ],
    )
    def _pack(tabT, w_hbm, src_v, stg_v, rs0, rs1, ws0, ws1):
        wid = lax.axis_index("s") * NC + lax.axis_index("c")
        kcount = jnp.where(wid < extra, per_sub + 1, per_sub)
        rsems = (rs0, rs1)
        wsems = (ws0, ws1)
        iota16 = lax.iota(jnp.int32, 16)

        def rd_start(j, b):
            off = pl.multiple_of(j * _BK, _BK)
            pltpu.async_copy(tabT.at[:, pl.ds(off, _BK)], src_v.at[b], rsems[b])

        def rd_wait(b):
            pltpu.make_async_copy(
                tabT.at[:, pl.ds(0, _BK)], src_v.at[b], rsems[b]
            ).wait()

        def wr_start(j, b):
            off = pl.multiple_of(j * wrows_blk, wrows_blk)
            pltpu.async_copy(
                stg_v.at[b], w_hbm.at[pl.ds(off, wrows_blk)], wsems[b]
            )

        def wr_wait(b):
            pltpu.make_async_copy(
                stg_v.at[b], w_hbm.at[pl.ds(0, wrows_blk)], wsems[b]
            ).wait()

        def transpose_block(b):
            @plsc.parallel_loop(
                0, _BK, unroll=32, carry=jnp.zeros((16,), jnp.int32)
            )
            def _tr(i, col):
                v = plsc.load_gather(src_v.at[b], [iota16, col])
                stg_v[b, i >> 3, pl.ds((i & 7) * 16, 16)] = v
                return col + 1

        rd_start(wid, 0)

        def body(k, carry):
            j = wid + NW * k
            for b in range(2):
                @pl.when(k % 2 == b)
                def _(b=b, j=j, k=k):
                    rd_wait(b)

                    @pl.when(k + 1 < kcount)
                    def _():
                        rd_start(j + NW, 1 - b)

                    @pl.when(k >= 2)
                    def _():
                        wr_wait(b)

                    transpose_block(b)
                    wr_start(j, b)

            return carry

        lax.fori_loop(0, kcount, body, 0)

        @pl.when(kcount >= 2)
        def _():
            wr_wait(0)
            wr_wait(1)

        @pl.when(kcount == 1)
        def _():
            wr_wait(0)

    b_per_sub = B // NW

    @functools.partial(
        pl.kernel,
        mesh=mesh,
        compiler_params=pltpu.CompilerParams(
            use_tc_tiling_on_sc=False, needs_layout_passes=False
        ),
        out_type=jax.ShapeDtypeStruct((L, D, B), jnp.float32),
        scratch_types=[
            pltpu.VMEM((L, b_per_sub), jnp.int32),
            pltpu.VMEM((2, b_per_sub, D), jnp.float32),
            pltpu.VMEM((D, b_per_sub), jnp.float32),
            pltpu.VMEM((tail, D), jnp.float32),
            pltpu.SemaphoreType.DMA,
            pltpu.SemaphoreType.DMA,
        ],
    )
    def _gath(w16, idxT, tail_hbm, outT, idx_v, rows_v, stg_v, tail_v, g0, g1):
        wid = lax.axis_index("s") * NC + lax.axis_index("c")
        b0 = wid * b_per_sub
        gsems = (g0, g1)
        iota16 = lax.iota(jnp.int32, 16)

        pltpu.sync_copy(idxT.at[:, pl.ds(b0, b_per_sub)], idx_v)
        pltpu.sync_copy(tail_hbm, tail_v)

        def g_start(l, b):
            pltpu.async_copy(w16.at[idx_v.at[l]], rows_v.at[b], gsems[b])

        def g_wait(b):
            pltpu.make_async_copy(
                w16.at[pl.ds(0, b_per_sub)], rows_v.at[b], gsems[b]
            ).wait()

        g_start(0, 0)
        g_start(1, 1)

        def body(i, carry):
            for b in range(2):
                l = 2 * i + b
                g_wait(b)
                for m in range(b_per_sub // 16):
                    iv = idx_v[l, pl.ds(16 * m, 16)]
                    msk = iv >= n_full * _BK

                    @pl.when(jnp.max(iv) >= n_full * _BK)
                    def _(iv=iv, msk=msk, m=m, b=b):
                        ti = jnp.where(msk, iv - n_full * _BK, 0)
                        for c in range(D):
                            tv = plsc.load_gather(
                                tail_v, [ti, jnp.full((16,), c, jnp.int32)]
                            )
                            plsc.store_scatter(
                                rows_v.at[b],
                                [iota16 + 16 * m,
                                 jnp.full((16,), c, jnp.int32)],
                                tv, mask=msk,
                            )

                @plsc.parallel_loop(0, D * b_per_sub // 16, unroll=16)
                def _tp(t, b=b):
                    v = plsc.load_gather(
                        rows_v.at[b],
                        [iota16 + (t & 0x70), jnp.full((16,), 0, jnp.int32)
                         + (t & 15)],
                    )
                    stg_v[t & 15, pl.ds((t >> 4) * 16, 16)] = v

                @pl.when(l + 2 < L)
                def _(l=l, b=b):
                    g_start(l + 2, b)

                pltpu.sync_copy(stg_v, outT.at[l, :, pl.ds(b0, b_per_sub)])
            return carry

        lax.fori_loop(0, L // 2, body, 0)

    w = _pack(values.T)
    outT = _gath(w.reshape(V, D), indices.T, values[n_full * _BK:, :])
    return jnp.transpose(outT, (2, 0, 1))


# speedup vs baseline: 2.3259x; 1.0024x over previous
import functools

import jax
import jax.numpy as jnp
from jax import lax
from jax.experimental import pallas as pl
from jax.experimental.pallas import tpu as pltpu
from jax.experimental.pallas import tpu_sc as plsc

_BK = 1024


def kernel(indices, values):
    B, L = indices.shape
    V, D = values.shape

    info = plsc.get_sparse_core_info()
    NC, NS = info.num_cores, info.num_subcores
    NW = NC * NS

    n_full = V // _BK
    tail = V - n_full * _BK
    wrows_blk = _BK * D // 128
    per_sub = n_full // NW
    extra = n_full - per_sub * NW

    mesh = plsc.VectorSubcoreMesh(core_axis_name="c", subcore_axis_name="s")

    @functools.partial(
        pl.kernel,
        mesh=mesh,
        compiler_params=pltpu.CompilerParams(needs_layout_passes=False),
        out_type=jax.ShapeDtypeStruct((V * D // 128, 128), jnp.float32),
        scratch_types=[
            pltpu.VMEM((2, D, _BK + 1), jnp.float32),
            pltpu.VMEM((2, wrows_blk, 128), jnp.float32),
            pltpu.SemaphoreType.DMA,
            pltpu.SemaphoreType.DMA,
            pltpu.SemaphoreType.DMA,
            pltpu.SemaphoreType.DMA,
        ],
    )
    def _pack(tabT, w_hbm, src_v, stg_v, rs0, rs1, ws0, ws1):
        wid = lax.axis_index("s") * NC + lax.axis_index("c")
        kcount = jnp.where(wid < extra, per_sub + 1, per_sub)
        rsems = (rs0, rs1)
        wsems = (ws0, ws1)
        iota16 = lax.iota(jnp.int32, 16)

        def rd_start(j, b):
            off = pl.multiple_of(j * _BK, _BK)
            pltpu.async_copy(
                tabT.at[:, pl.ds(off, _BK)],
                src_v.at[b, :, pl.ds(0, _BK)], rsems[b],
            )

        def rd_wait(b):
            pltpu.make_async_copy(
                tabT.at[:, pl.ds(0, _BK)],
                src_v.at[b, :, pl.ds(0, _BK)], rsems[b],
            ).wait()

        def wr_start(j, b):
            off = pl.multiple_of(j * wrows_blk, wrows_blk)
            pltpu.async_copy(
                stg_v.at[b], w_hbm.at[pl.ds(off, wrows_blk)], wsems[b]
            )

        def wr_wait(b):
            pltpu.make_async_copy(
                stg_v.at[b], w_hbm.at[pl.ds(0, wrows_blk)], wsems[b]
            ).wait()

        def transpose_block(b):
            @plsc.parallel_loop(
                0, _BK, unroll=32, carry=jnp.zeros((16,), jnp.int32)
            )
            def _tr(i, col):
                v = plsc.load_gather(src_v.at[b], [iota16, col])
                stg_v[b, i >> 3, pl.ds((i & 7) * 16, 16)] = v
                return col + 1

        rd_start(wid, 0)

        def body(k, carry):
            j = wid + NW * k
            for b in range(2):
                @pl.when(k % 2 == b)
                def _(b=b, j=j, k=k):
                    rd_wait(b)

                    @pl.when(k + 1 < kcount)
                    def _():
                        rd_start(j + NW, 1 - b)

                    @pl.when(k >= 2)
                    def _():
                        wr_wait(b)

                    transpose_block(b)
                    wr_start(j, b)

            return carry

        lax.fori_loop(0, kcount, body, 0)

        @pl.when(kcount >= 2)
        def _():
            wr_wait(0)
            wr_wait(1)

        @pl.when(kcount == 1)
        def _():
            wr_wait(0)

    b_per_sub = B // NW

    @functools.partial(
        pl.kernel,
        mesh=mesh,
        compiler_params=pltpu.CompilerParams(
            use_tc_tiling_on_sc=False, needs_layout_passes=False
        ),
        out_type=jax.ShapeDtypeStruct((L, D, B), jnp.float32),
        scratch_types=[
            pltpu.VMEM((L, b_per_sub), jnp.int32),
            pltpu.VMEM((2, b_per_sub, D), jnp.float32),
            pltpu.VMEM((D, b_per_sub), jnp.float32),
            pltpu.VMEM((tail, D), jnp.float32),
            pltpu.SemaphoreType.DMA,
            pltpu.SemaphoreType.DMA,
        ],
    )
    def _gath(w16, idxT, tail_hbm, outT, idx_v, rows_v, stg_v, tail_v, g0, g1):
        wid = lax.axis_index("s") * NC + lax.axis_index("c")
        b0 = wid * b_per_sub
        gsems = (g0, g1)
        iota16 = lax.iota(jnp.int32, 16)

        pltpu.sync_copy(idxT.at[:, pl.ds(b0, b_per_sub)], idx_v)
        pltpu.sync_copy(tail_hbm, tail_v)

        def g_start(l, b):
            pltpu.async_copy(w16.at[idx_v.at[l]], rows_v.at[b], gsems[b])

        def g_wait(b):
            pltpu.make_async_copy(
                w16.at[pl.ds(0, b_per_sub)], rows_v.at[b], gsems[b]
            ).wait()

        g_start(0, 0)
        g_start(1, 1)

        def body(i, carry):
            for b in range(2):
                l = 2 * i + b
                g_wait(b)
                for m in range(b_per_sub // 16):
                    iv = idx_v[l, pl.ds(16 * m, 16)]
                    msk = iv >= n_full * _BK

                    @pl.when(jnp.max(iv) >= n_full * _BK)
                    def _(iv=iv, msk=msk, m=m, b=b):
                        ti = jnp.where(msk, iv - n_full * _BK, 0)
                        for c in range(D):
                            tv = plsc.load_gather(
                                tail_v, [ti, jnp.full((16,), c, jnp.int32)]
                            )
                            plsc.store_scatter(
                                rows_v.at[b],
                                [iota16 + 16 * m,
                                 jnp.full((16,), c, jnp.int32)],
                                tv, mask=msk,
                            )

                @plsc.parallel_loop(0, D * b_per_sub // 16, unroll=16)
                def _tp(t, b=b):
                    v = plsc.load_gather(
                        rows_v.at[b],
                        [iota16 + (t & 0x70), jnp.full((16,), 0, jnp.int32)
                         + (t & 15)],
                    )
                    stg_v[t & 15, pl.ds((t >> 4) * 16, 16)] = v

                @pl.when(l + 2 < L)
                def _(l=l, b=b):
                    g_start(l + 2, b)

                pltpu.sync_copy(stg_v, outT.at[l, :, pl.ds(b0, b_per_sub)])
            return carry

        lax.fori_loop(0, L // 2, body, 0)

    w = _pack(values.T)
    outT = _gath(w.reshape(V, D), indices.T, values[n_full * _BK:, :])
    return jnp.transpose(outT, (2, 0, 1))
